# baseline (speedup 1.0000x reference)
import jax
import jax.numpy as jnp
from jax import lax
from jax.experimental import pallas as pl
from jax.experimental.pallas import tpu as pltpu

_ROWS = 4 * 4096
_D = 1024
_K = _ROWS // 2
_BLK = 2048
_NBLK = _ROWS // _BLK


def _loss_body(pred_ref, target_ref, loss_ref):
    d = jnp.abs(pred_ref[...] - target_ref[...])
    part = d[:, 0:128]
    for j in range(1, _D // 128):
        part = part + d[:, j * 128:(j + 1) * 128]
    loss_ref[0, 0, :] = jnp.sum(part, axis=1)


def _select_body(loss_ref, out_ref):
    v = loss_ref[...]
    bits = lax.bitcast_convert_type(v, jnp.int32)

    def step(_, carry):
        lo, hi = carry
        mid = lo + (hi - lo + 1) // 2
        cnt = jnp.sum((bits >= mid).astype(jnp.int32))
        ok = cnt >= _K
        return jnp.where(ok, mid, lo), jnp.where(ok, hi, mid - 1)

    lo, _hi = lax.fori_loop(
        0, 31, step, (jnp.int32(0), jnp.int32(0x7F7FFFFF)))
    t = lax.bitcast_convert_type(lo, jnp.float32)
    gt = bits > lo
    m = jnp.sum(gt.astype(jnp.int32)).astype(jnp.float32)
    sum_gt = jnp.sum(jnp.where(gt, v, 0.0))
    total = sum_gt + (jnp.float32(_K) - m) * t
    out_ref[...] = jnp.full((1, 1), total / jnp.float32(_K), jnp.float32)


def kernel(pred, target):
    p = pred.reshape(_ROWS, _D)
    t = target.reshape(_ROWS, _D)
    losses = pl.pallas_call(
        _loss_body,
        grid=(_NBLK,),
        in_specs=[
            pl.BlockSpec((_BLK, _D), lambda i: (i, 0)),
            pl.BlockSpec((_BLK, _D), lambda i: (i, 0)),
        ],
        out_specs=pl.BlockSpec((1, 1, _BLK), lambda i: (i, 0, 0)),
        out_shape=jax.ShapeDtypeStruct((_NBLK, 1, _BLK), jnp.float32),
        compiler_params=pltpu.CompilerParams(
            dimension_semantics=("parallel",)),
    )(p, t)
    out = pl.pallas_call(
        _select_body,
        out_shape=jax.ShapeDtypeStruct((1, 1), jnp.float32),
    )(losses)
    return out[0, 0]


# speedup vs baseline: 1.1045x; 1.1045x over previous
import jax
import jax.numpy as jnp
from jax import lax
from jax.experimental import pallas as pl
from jax.experimental.pallas import tpu as pltpu

_ROWS = 4 * 4096
_D = 1024
_K = _ROWS // 2
_BLK = 2048
_NBLK = _ROWS // _BLK


def _topk_l1_body(pred_ref, target_ref, out_ref, loss_ref):
    i = pl.program_id(0)
    d = jnp.abs(pred_ref[...] - target_ref[...])
    part = d[:, 0:128]
    for j in range(1, _D // 128):
        part = part + d[:, j * 128:(j + 1) * 128]
    loss_ref[i, :] = jnp.sum(part, axis=1)

    @pl.when(i == _NBLK - 1)
    def _():
        v = loss_ref[...]
        bits = lax.bitcast_convert_type(v, jnp.int32)

        def cond(carry):
            lo, hi = carry
            return lo < hi

        def step(carry):
            lo, hi = carry
            mid = lo + (hi - lo + 1) // 2
            cnt = jnp.sum((bits >= mid).astype(jnp.int32))
            ok = cnt >= _K
            return jnp.where(ok, mid, lo), jnp.where(ok, hi, mid - 1)

        lo0 = jnp.min(bits)
        hi0 = jnp.max(bits)
        lo, _hi = lax.while_loop(cond, step, (lo0, hi0))
        t = lax.bitcast_convert_type(lo, jnp.float32)
        gt = bits > lo
        m = jnp.sum(gt.astype(jnp.int32)).astype(jnp.float32)
        sum_gt = jnp.sum(jnp.where(gt, v, 0.0))
        total = sum_gt + (jnp.float32(_K) - m) * t
        out_ref[...] = jnp.full((1, 1), total / jnp.float32(_K), jnp.float32)


def kernel(pred, target):
    p = pred.reshape(_ROWS, _D)
    t = target.reshape(_ROWS, _D)
    out = pl.pallas_call(
        _topk_l1_body,
        grid=(_NBLK,),
        in_specs=[
            pl.BlockSpec((_BLK, _D), lambda i: (i, 0)),
            pl.BlockSpec((_BLK, _D), lambda i: (i, 0)),
        ],
        out_specs=pl.BlockSpec((1, 1), lambda i: (0, 0)),
        out_shape=jax.ShapeDtypeStruct((1, 1), jnp.float32),
        scratch_shapes=[pltpu.VMEM((_NBLK, _BLK), jnp.float32)],
    )(p, t)
    return out[0, 0]


# speedup vs baseline: 1.1693x; 1.0586x over previous
import jax
import jax.numpy as jnp
from jax import lax
from jax.experimental import pallas as pl
from jax.experimental.pallas import tpu as pltpu

_ROWS = 4 * 4096
_D = 1024
_K = _ROWS // 2
_BLK = 1024
_NBLK = _ROWS // _BLK
_WAYS = 8


def _topk_l1_body(pred_ref, target_ref, out_ref, loss_ref):
    i = pl.program_id(0)
    d = jnp.abs(pred_ref[...] - target_ref[...])
    part = d[:, 0:128]
    for j in range(1, _D // 128):
        part = part + d[:, j * 128:(j + 1) * 128]
    loss_ref[i, :] = jnp.sum(part, axis=1)

    @pl.when(i == _NBLK - 1)
    def _():
        v = loss_ref[...]
        bits = lax.bitcast_convert_type(v, jnp.int32)

        def cond(carry):
            lo, hi = carry
            return lo < hi

        def round_(carry):
            lo, hi = carry
            w = hi - lo + 1
            step = jnp.maximum(w // _WAYS, 1)
            new_lo, new_hi = lo, hi
            for j in range(1, _WAYS):
                p = lo + j * step
                cnt = jnp.sum((bits >= p).astype(jnp.int32))
                ok = cnt >= _K
                new_lo = jnp.where(ok, p, new_lo)
                new_hi = jnp.where(ok, new_hi, jnp.minimum(new_hi, p - 1))
            return new_lo, new_hi

        lo0 = jnp.min(bits)
        hi0 = jnp.max(bits)
        lo, _hi = lax.while_loop(cond, round_, (lo0, hi0))
        t = lax.bitcast_convert_type(lo, jnp.float32)
        gt = bits > lo
        m = jnp.sum(gt.astype(jnp.int32)).astype(jnp.float32)
        sum_gt = jnp.sum(jnp.where(gt, v, 0.0))
        total = sum_gt + (jnp.float32(_K) - m) * t
        out_ref[...] = jnp.full((1, 1), total / jnp.float32(_K), jnp.float32)


def kernel(pred, target):
    p = pred.reshape(_ROWS, _D)
    t = target.reshape(_ROWS, _D)
    out = pl.pallas_call(
        _topk_l1_body,
        grid=(_NBLK,),
        in_specs=[
            pl.BlockSpec((_BLK, _D), lambda i: (i, 0)),
            pl.BlockSpec((_BLK, _D), lambda i: (i, 0)),
        ],
        out_specs=pl.BlockSpec((1, 1), lambda i: (0, 0)),
        out_shape=jax.ShapeDtypeStruct((1, 1), jnp.float32),
        scratch_shapes=[pltpu.VMEM((_NBLK, _BLK), jnp.float32)],
    )(p, t)
    return out[0, 0]
